# initial kernel scaffold (unmeasured)
import jax
import jax.numpy as jnp
from jax import lax
from jax.experimental import pallas as pl
from jax.experimental.pallas import tpu as pltpu

N_DEV = 8


def _gelu(y):
    c = 0.7978845608028654
    return 0.5 * y * (1.0 + jnp.tanh(c * (y + 0.044715 * y * y * y)))


def kernel(x, w_mat):
    m_tot, _ = x.shape
    _, n = w_mat.shape
    m_per = m_tot // N_DEV

    def body(x_ref, w_ref, out_ref, send_ref, recv_ref,
             send_sems, recv_sems, credit_sem):
        my = lax.axis_index("i")
        left = lax.rem(my + N_DEV - 1, N_DEV)
        right = lax.rem(my + 1, N_DEV)

        barrier = pltpu.get_barrier_semaphore()
        for nbr in (left, right):
            pl.semaphore_signal(barrier, inc=1, device_id=(nbr,),
                                device_id_type=pl.DeviceIdType.MESH)
        pl.semaphore_wait(barrier, 2)

        def partial(c):
            xc = x_ref[pl.ds(c * m_per, m_per), :]
            return jnp.dot(xc, w_ref[...], preferred_element_type=jnp.float32)

        def send(slot):
            rdma = pltpu.make_async_remote_copy(
                src_ref=send_ref,
                dst_ref=recv_ref.at[slot],
                send_sem=send_sems.at[slot],
                recv_sem=recv_sems.at[slot],
                device_id=(right,),
                device_id_type=pl.DeviceIdType.MESH,
            )
            rdma.start()
            rdma.wait()

        send_ref[...] = partial(lax.rem(my + N_DEV - 1, N_DEV))
        send(0)

        for t in range(1, N_DEV - 1):
            prev_slot = (t - 1) % 2
            c = lax.rem(my + N_DEV - 1 - t, N_DEV)
            send_ref[...] = recv_ref[prev_slot] + partial(c)
            if t <= N_DEV - 3:
                pl.semaphore_signal(credit_sem, inc=1, device_id=(left,),
                                    device_id_type=pl.DeviceIdType.MESH)
            if t >= 2:
                pl.semaphore_wait(credit_sem, 1)
            send(t % 2)

        out_ref[...] = _gelu(recv_ref[(N_DEV - 2) % 2] + partial(my))

    return pl.pallas_call(
        body,
        out_shape=jax.ShapeDtypeStruct((m_per, n), jnp.float32),
        in_specs=[
            pl.BlockSpec(memory_space=pltpu.VMEM),
            pl.BlockSpec(memory_space=pltpu.VMEM),
        ],
        out_specs=pl.BlockSpec(memory_space=pltpu.VMEM),
        scratch_shapes=[
            pltpu.VMEM((m_per, n), jnp.float32),
            pltpu.VMEM((2, m_per, n), jnp.float32),
            pltpu.SemaphoreType.DMA((2,)),
            pltpu.SemaphoreType.DMA((2,)),
            pltpu.SemaphoreType.REGULAR,
        ],
        compiler_params=pltpu.CompilerParams(collective_id=0),
    )(x, w_mat)


# baseline (device time: 1379450 ns/iter reference)
import jax
import jax.numpy as jnp
from jax import lax
from jax.experimental import pallas as pl
from jax.experimental.pallas import tpu as pltpu

N_DEV = 8
N_HALF = 2


def _gelu(y):
    c = 0.7978845608028654
    return 0.5 * y * (1.0 + jnp.tanh(c * (y + 0.044715 * y * y * y)))


def kernel(x, w_mat):
    m_tot, _ = x.shape
    _, n = w_mat.shape
    m_per = m_tot // N_DEV
    n_h = n // N_HALF

    def body(x_ref, w_ref, out_ref, send_ref, recv_ref,
             send_sem, recv_sem, out_sem, credit_sem):
        my = lax.axis_index("i")
        left = lax.rem(my + N_DEV - 1, N_DEV)
        right = lax.rem(my + 1, N_DEV)

        barrier = pltpu.get_barrier_semaphore()
        for nbr in (left, right):
            pl.semaphore_signal(barrier, inc=1, device_id=(nbr,),
                                device_id_type=pl.DeviceIdType.MESH)
        pl.semaphore_wait(barrier, 2)

        def partial(c, jh):
            xc = x_ref[pl.ds(c * m_per, m_per), :]
            wh = w_ref[:, jh * n_h:(jh + 1) * n_h]
            return jnp.dot(xc, wh, preferred_element_type=jnp.float32)

        def send():
            rdma = pltpu.make_async_remote_copy(
                src_ref=send_ref,
                dst_ref=recv_ref,
                send_sem=send_sem,
                recv_sem=recv_sem,
                device_id=(right,),
                device_id_type=pl.DeviceIdType.MESH,
            )
            rdma.start()
            rdma.wait()

        for jh in range(N_HALF):
            for t in range(N_DEV - 1):
                c = lax.rem(my + N_DEV - 1 - t, N_DEV)
                if t == 0:
                    send_ref[...] = partial(c, jh)
                else:
                    send_ref[...] = recv_ref[...] + partial(c, jh)
                    pl.semaphore_signal(credit_sem, inc=1, device_id=(left,),
                                        device_id_type=pl.DeviceIdType.MESH)
                if not (jh == 0 and t == 0):
                    pl.semaphore_wait(credit_sem, 1)
                send()

            send_ref[...] = _gelu(recv_ref[...] + partial(my, jh))
            if jh < N_HALF - 1:
                pl.semaphore_signal(credit_sem, inc=1, device_id=(left,),
                                    device_id_type=pl.DeviceIdType.MESH)
            out_copy = pltpu.make_async_copy(
                send_ref, out_ref.at[:, jh * n_h:(jh + 1) * n_h], out_sem)
            out_copy.start()
            out_copy.wait()

    return pl.pallas_call(
        body,
        out_shape=jax.ShapeDtypeStruct((m_per, n), jnp.float32),
        in_specs=[
            pl.BlockSpec(memory_space=pltpu.VMEM),
            pl.BlockSpec(memory_space=pltpu.VMEM),
        ],
        out_specs=pl.BlockSpec(memory_space=pl.ANY),
        scratch_shapes=[
            pltpu.VMEM((m_per, n_h), jnp.float32),
            pltpu.VMEM((m_per, n_h), jnp.float32),
            pltpu.SemaphoreType.DMA,
            pltpu.SemaphoreType.DMA,
            pltpu.SemaphoreType.DMA,
            pltpu.SemaphoreType.REGULAR,
        ],
        compiler_params=pltpu.CompilerParams(
            collective_id=0,
            vmem_limit_bytes=60 * 1024 * 1024,
        ),
    )(x, w_mat)


# device time: 1353640 ns/iter; 1.0191x vs baseline; 1.0191x over previous
import jax
import jax.numpy as jnp
from jax import lax
from jax.experimental import pallas as pl
from jax.experimental.pallas import tpu as pltpu

N_DEV = 8
N_HALF = 2


def _gelu(y):
    c = 0.7978845608028654
    return 0.5 * y * (1.0 + jnp.tanh(c * (y + 0.044715 * y * y * y)))


def kernel(x, w_mat):
    m_tot, _ = x.shape
    _, n = w_mat.shape
    m_per = m_tot // N_DEV
    n_h = n // N_HALF

    def body(x_ref, w_ref, out_ref, send_ref, recv_ref,
             send_sem, recv_sem, out_sem, credit_sem):
        my = lax.axis_index("i")
        left = lax.rem(my + N_DEV - 1, N_DEV)
        right = lax.rem(my + 1, N_DEV)

        barrier = pltpu.get_barrier_semaphore()
        for nbr in (left, right):
            pl.semaphore_signal(barrier, inc=1, device_id=(nbr,),
                                device_id_type=pl.DeviceIdType.MESH)
        pl.semaphore_wait(barrier, 2)

        def partial(c, jh):
            xc = x_ref[pl.ds(c * m_per, m_per), :]
            wh = w_ref[:, jh * n_h:(jh + 1) * n_h]
            return jnp.dot(xc, wh, preferred_element_type=jnp.float32)

        def make_rdma():
            return pltpu.make_async_remote_copy(
                src_ref=send_ref,
                dst_ref=recv_ref,
                send_sem=send_sem,
                recv_sem=recv_sem,
                device_id=(right,),
                device_id_type=pl.DeviceIdType.MESH,
            )

        pending = None
        for jh in range(N_HALF):
            for t in range(N_DEV - 1):
                c = lax.rem(my + N_DEV - 1 - t, N_DEV)
                p = partial(c, jh)
                if t == 0:
                    send_ref[...] = p
                else:
                    pending.wait()
                    send_ref[...] = recv_ref[...] + p
                    pl.semaphore_signal(credit_sem, inc=1, device_id=(left,),
                                        device_id_type=pl.DeviceIdType.MESH)
                if not (jh == 0 and t == 0):
                    pl.semaphore_wait(credit_sem, 1)
                pending = make_rdma()
                pending.start()

            p = partial(my, jh)
            pending.wait()
            pending = None
            send_ref[...] = _gelu(recv_ref[...] + p)
            if jh < N_HALF - 1:
                pl.semaphore_signal(credit_sem, inc=1, device_id=(left,),
                                    device_id_type=pl.DeviceIdType.MESH)
            out_copy = pltpu.make_async_copy(
                send_ref, out_ref.at[:, jh * n_h:(jh + 1) * n_h], out_sem)
            out_copy.start()
            out_copy.wait()

    return pl.pallas_call(
        body,
        out_shape=jax.ShapeDtypeStruct((m_per, n), jnp.float32),
        in_specs=[
            pl.BlockSpec(memory_space=pltpu.VMEM),
            pl.BlockSpec(memory_space=pltpu.VMEM),
        ],
        out_specs=pl.BlockSpec(memory_space=pl.ANY),
        scratch_shapes=[
            pltpu.VMEM((m_per, n_h), jnp.float32),
            pltpu.VMEM((m_per, n_h), jnp.float32),
            pltpu.SemaphoreType.DMA,
            pltpu.SemaphoreType.DMA,
            pltpu.SemaphoreType.DMA,
            pltpu.SemaphoreType.REGULAR,
        ],
        compiler_params=pltpu.CompilerParams(
            collective_id=0,
            vmem_limit_bytes=60 * 1024 * 1024,
        ),
    )(x, w_mat)


# device time: 697227 ns/iter; 1.9785x vs baseline; 1.9415x over previous
import jax
import jax.numpy as jnp
from jax import lax
from jax.experimental import pallas as pl
from jax.experimental.pallas import tpu as pltpu

N_DEV = 8
N_PASS = 2


def _gelu(y):
    c = 0.7978845608028654
    return 0.5 * y * (1.0 + jnp.tanh(c * (y + 0.044715 * y * y * y)))


def kernel(x, w_mat):
    m_tot, _ = x.shape
    _, n = w_mat.shape
    m_per = m_tot // N_DEV
    n_q = n // 4

    def body(x_ref, w_ref, out_ref, sendR, sendL, recvR, recvL,
             send_semR, send_semL, recv_semsR, recv_semsL,
             out_semR, out_semL, creditR, creditL):
        my = lax.axis_index("i")
        left = lax.rem(my + N_DEV - 1, N_DEV)
        right = lax.rem(my + 1, N_DEV)

        barrier = pltpu.get_barrier_semaphore()
        for nbr in (left, right):
            pl.semaphore_signal(barrier, inc=1, device_id=(nbr,),
                                device_id_type=pl.DeviceIdType.MESH)
        pl.semaphore_wait(barrier, 2)

        def partial(c, q):
            xc = x_ref[pl.ds(c * m_per, m_per), :]
            wq = w_ref[:, q * n_q:(q + 1) * n_q]
            return jnp.dot(xc, wq, preferred_element_type=jnp.float32)

        def rdma(src, dst_slots, slot, send_sem, recv_sems, to):
            return pltpu.make_async_remote_copy(
                src_ref=src,
                dst_ref=dst_slots.at[slot],
                send_sem=send_sem,
                recv_sem=recv_sems.at[slot],
                device_id=(to,),
                device_id_type=pl.DeviceIdType.MESH,
            )

        rings = (
            (sendR, recvR, send_semR, recv_semsR, right, left, creditR, out_semR),
            (sendL, recvL, send_semL, recv_semsL, left, right, creditL, out_semL),
        )
        pending = [None, None]

        for jq in range(N_PASS):
            for t in range(N_DEV - 1):
                k = jq * (N_DEV - 1) + t
                cR = lax.rem(my + N_DEV - 1 - t, N_DEV)
                cL = lax.rem(my + 1 + t, N_DEV)
                parts = (partial(cR, jq), partial(cL, 2 + jq))
                for r, (sbuf, rslots, ssem, rsems, to, upstream, credit,
                        _osem) in enumerate(rings):
                    p = parts[r]
                    if t == 0:
                        sbuf[...] = p
                    else:
                        pending[r].wait()
                        sbuf[...] = rslots[(k - 1) % 2] + p
                        if k <= 12:
                            pl.semaphore_signal(
                                credit, inc=1, device_id=(upstream,),
                                device_id_type=pl.DeviceIdType.MESH)
                    if k >= 2:
                        pl.semaphore_wait(credit, 1)
                    pending[r] = rdma(sbuf, rslots, k % 2, ssem, rsems, to)
                    pending[r].start()

            k_last = jq * (N_DEV - 1) + N_DEV - 2
            parts = (partial(my, jq), partial(my, 2 + jq))
            copies = []
            for r, (sbuf, rslots, _ssem, _rsems, _to, upstream, credit,
                    osem) in enumerate(rings):
                q = (jq, 2 + jq)[r]
                pending[r].wait()
                pending[r] = None
                sbuf[...] = _gelu(rslots[k_last % 2] + parts[r])
                if jq < N_PASS - 1:
                    pl.semaphore_signal(
                        credit, inc=1, device_id=(upstream,),
                        device_id_type=pl.DeviceIdType.MESH)
                cp = pltpu.make_async_copy(
                    sbuf, out_ref.at[:, q * n_q:(q + 1) * n_q], osem)
                cp.start()
                copies.append(cp)
            for cp in copies:
                cp.wait()

    return pl.pallas_call(
        body,
        out_shape=jax.ShapeDtypeStruct((m_per, n), jnp.float32),
        in_specs=[
            pl.BlockSpec(memory_space=pltpu.VMEM),
            pl.BlockSpec(memory_space=pltpu.VMEM),
        ],
        out_specs=pl.BlockSpec(memory_space=pl.ANY),
        scratch_shapes=[
            pltpu.VMEM((m_per, n_q), jnp.float32),
            pltpu.VMEM((m_per, n_q), jnp.float32),
            pltpu.VMEM((2, m_per, n_q), jnp.float32),
            pltpu.VMEM((2, m_per, n_q), jnp.float32),
            pltpu.SemaphoreType.DMA,
            pltpu.SemaphoreType.DMA,
            pltpu.SemaphoreType.DMA((2,)),
            pltpu.SemaphoreType.DMA((2,)),
            pltpu.SemaphoreType.DMA,
            pltpu.SemaphoreType.DMA,
            pltpu.SemaphoreType.REGULAR,
            pltpu.SemaphoreType.REGULAR,
        ],
        compiler_params=pltpu.CompilerParams(
            collective_id=0,
            vmem_limit_bytes=60 * 1024 * 1024,
        ),
    )(x, w_mat)
